# baseline (speedup 1.0000x reference)
import jax
import jax.numpy as jnp

DELTA = 2.5
EIG_IDX = 1


def _bn(h, g, b, eps=1e-5):
    mu = jnp.mean(h, axis=0, keepdims=True)
    var = jnp.var(h, axis=0, keepdims=True)
    return g * (h - mu) * jax.lax.rsqrt(var + eps) + b


def _conv(h, eig, p, src, dst):
    n = h.shape[0]
    f = jnp.concatenate([h[src], h[dst]], axis=-1)
    msg = f @ p['M_W'] + p['M_b']
    diff = eig[src, EIG_IDX] - eig[dst, EIG_IDX]
    abs_diff = jnp.abs(diff)
    norm = jax.ops.segment_sum(abs_diff, dst, num_segments=n)
    inv = 1.0 / (norm[dst] + 1e-30)
    w_av = (abs_diff * inv)[:, None]
    h_av = jax.ops.segment_sum(msg * w_av, dst, num_segments=n)
    w_dx = (diff * inv)[:, None]
    h_dx = jnp.abs(
        jax.ops.segment_sum(msg * w_dx, dst, num_segments=n)
        - jax.ops.segment_sum(w_dx, dst, num_segments=n) * h
    )
    h_sum = jax.ops.segment_sum(msg, dst, num_segments=n)
    h_agg = jnp.concatenate([h_av, h_dx, h_sum], axis=-1)
    deg = jax.ops.segment_sum(jnp.ones_like(abs_diff), dst, num_segments=n)
    amp = (jnp.log(deg + 1.0) / DELTA)[:, None]
    h_scaled = jnp.concatenate([h_agg, h_agg * amp], axis=-1)
    h_out = jnp.concatenate([h, h_scaled], axis=-1) @ p['U_W'] + p['U_b']
    h_out = jax.nn.leaky_relu(h_out @ p['mix_W'] + p['mix_b'], negative_slope=0.01)
    return h_out + h


def kernel(x, edge_index, eig, params):
    src, dst = edge_index[0], edge_index[1]
    h = jax.nn.relu(_bn(x @ params['mlp_W1'] + params['mlp_b1'],
                        params['mlp_g1'], params['mlp_be1']))
    h = jax.nn.relu(_bn(h @ params['mlp_W2'] + params['mlp_b2'],
                        params['mlp_g2'], params['mlp_be2']))
    h = h @ params['mlp_W3'] + params['mlp_b3']
    for p in params['convs']:
        h = _bn(_conv(h, eig, p, src, dst), p['bn_g'], p['bn_b'])
    return jnp.sum(h, axis=0, keepdims=True)


# speedup vs baseline: 1.1732x; 1.1732x over previous
import functools

import jax
import jax.numpy as jnp
from jax import lax
from jax.experimental import pallas as pl
from jax.experimental.pallas import tpu as pltpu
from jax.experimental.pallas import tpu_sc as plsc

DELTA = 2.5
EIG_IDX = 1

N = 10000
E = 320000


def _mm_bias(a, w, b, bm):
    m, k = a.shape
    f = w.shape[1]

    def body(a_ref, w_ref, b_ref, o_ref):
        o_ref[...] = jnp.dot(a_ref[...], w_ref[...],
                             preferred_element_type=jnp.float32) + b_ref[...]

    return pl.pallas_call(
        body,
        grid=(m // bm,),
        in_specs=[pl.BlockSpec((bm, k), lambda i: (i, 0)),
                  pl.BlockSpec((k, f), lambda i: (0, 0)),
                  pl.BlockSpec((f,), lambda i: (0,))],
        out_specs=pl.BlockSpec((bm, f), lambda i: (i, 0)),
        out_shape=jax.ShapeDtypeStruct((m, f), jnp.float32),
    )(a, w, b)


def _mm_bias_leaky_res(a, w, b, h, bm):
    m, k = a.shape
    f = w.shape[1]

    def body(a_ref, w_ref, b_ref, h_ref, o_ref):
        z = jnp.dot(a_ref[...], w_ref[...],
                    preferred_element_type=jnp.float32) + b_ref[...]
        o_ref[...] = jax.nn.leaky_relu(z, negative_slope=0.01) + h_ref[...]

    return pl.pallas_call(
        body,
        grid=(m // bm,),
        in_specs=[pl.BlockSpec((bm, k), lambda i: (i, 0)),
                  pl.BlockSpec((k, f), lambda i: (0, 0)),
                  pl.BlockSpec((f,), lambda i: (0,)),
                  pl.BlockSpec((bm, f), lambda i: (i, 0))],
        out_specs=pl.BlockSpec((bm, f), lambda i: (i, 0)),
        out_shape=jax.ShapeDtypeStruct((m, f), jnp.float32),
    )(a, w, b, h)


def _bn_norm(y, g, b, mu, var, relu, bm):
    m, f = y.shape

    def body(y_ref, g_ref, b_ref, mu_ref, var_ref, o_ref):
        z = (g_ref[...] * (y_ref[...] - mu_ref[...])
             * lax.rsqrt(var_ref[...] + 1e-5) + b_ref[...])
        o_ref[...] = jax.nn.relu(z) if relu else z

    return pl.pallas_call(
        body,
        grid=(m // bm,),
        in_specs=[pl.BlockSpec((bm, f), lambda i: (i, 0)),
                  pl.BlockSpec((f,), lambda i: (0,)),
                  pl.BlockSpec((f,), lambda i: (0,)),
                  pl.BlockSpec((1, f), lambda i: (0, 0)),
                  pl.BlockSpec((1, f), lambda i: (0, 0))],
        out_specs=pl.BlockSpec((bm, f), lambda i: (i, 0)),
        out_shape=jax.ShapeDtypeStruct((m, f), jnp.float32),
    )(y, g, b, mu, var)


def _weighted_msgs(msg, w_av, w_dx, bm):
    m, f = msg.shape

    def body(m_ref, wa_ref, wd_ref, oa_ref, od_ref):
        mv = m_ref[...]
        oa_ref[...] = mv * wa_ref[...]
        od_ref[...] = mv * wd_ref[...]

    return pl.pallas_call(
        body,
        grid=(m // bm,),
        in_specs=[pl.BlockSpec((bm, f), lambda i: (i, 0)),
                  pl.BlockSpec((bm, 1), lambda i: (i, 0)),
                  pl.BlockSpec((bm, 1), lambda i: (i, 0))],
        out_specs=[pl.BlockSpec((bm, f), lambda i: (i, 0)),
                   pl.BlockSpec((bm, f), lambda i: (i, 0))],
        out_shape=[jax.ShapeDtypeStruct((m, f), jnp.float32),
                   jax.ShapeDtypeStruct((m, f), jnp.float32)],
    )(msg, w_av, w_dx)


_SC_INFO = plsc.get_sparse_core_info()
_NC, _NS = _SC_INFO.num_cores, _SC_INFO.num_subcores
_NW = _NC * _NS
_CH = 400


def _edge_features(table, src, dst):
    b_per_w = E // _NW
    mesh = plsc.VectorSubcoreMesh(core_axis_name="c", subcore_axis_name="s")

    @functools.partial(
        pl.kernel, mesh=mesh,
        out_type=jax.ShapeDtypeStruct((E, 256), jnp.float32),
        scratch_types=[
            pltpu.VMEM((_CH,), jnp.int32),
            pltpu.VMEM((_CH,), jnp.int32),
            pltpu.VMEM((_CH, 128), jnp.float32),
            pltpu.VMEM((_CH, 128), jnp.float32),
            pltpu.SemaphoreType.DMA,
            pltpu.SemaphoreType.DMA,
        ],
    )
    def k(table_hbm, src_hbm, dst_hbm, out_hbm, si_v, di_v, rs_v, rd_v, s1, s2):
        wid = lax.axis_index("s") * _NC + lax.axis_index("c")
        base = wid * b_per_w

        def step(j):
            off = base + j * _CH
            pltpu.sync_copy(src_hbm.at[pl.ds(off, _CH)], si_v)
            pltpu.sync_copy(dst_hbm.at[pl.ds(off, _CH)], di_v)
            cp1 = pltpu.async_copy(table_hbm.at[si_v], rs_v, s1)
            cp2 = pltpu.async_copy(table_hbm.at[di_v], rd_v, s2)
            cp1.wait()
            cp2.wait()
            pltpu.sync_copy(rs_v, out_hbm.at[pl.ds(off, _CH), pl.ds(0, 128)])
            pltpu.sync_copy(rd_v, out_hbm.at[pl.ds(off, _CH), pl.ds(128, 128)])

        pl.loop(0, b_per_w // _CH)(step)

    return k(table, src, dst)


def _conv(h, eig, p, src, dst):
    f = _edge_features(h, src, dst)
    msg = _mm_bias(f, p['M_W'], p['M_b'], bm=2000)
    diff = eig[src, EIG_IDX] - eig[dst, EIG_IDX]
    abs_diff = jnp.abs(diff)
    norm = jax.ops.segment_sum(abs_diff, dst, num_segments=N)
    inv = 1.0 / (norm[dst] + 1e-30)
    w_av = (abs_diff * inv)[:, None]
    w_dx = (diff * inv)[:, None]
    m_av, m_dx = _weighted_msgs(msg, w_av, w_dx, bm=2000)
    h_av = jax.ops.segment_sum(m_av, dst, num_segments=N)
    h_dx = jnp.abs(
        jax.ops.segment_sum(m_dx, dst, num_segments=N)
        - jax.ops.segment_sum(w_dx, dst, num_segments=N) * h
    )
    h_sum = jax.ops.segment_sum(msg, dst, num_segments=N)
    h_agg = jnp.concatenate([h_av, h_dx, h_sum], axis=-1)
    deg = jax.ops.segment_sum(jnp.ones_like(abs_diff), dst, num_segments=N)
    amp = (jnp.log(deg + 1.0) / DELTA)[:, None]
    h_scaled = jnp.concatenate([h_agg, h_agg * amp], axis=-1)
    h_out = jnp.concatenate([h, h_scaled], axis=-1) @ p['U_W'] + p['U_b']
    return _mm_bias_leaky_res(h_out, p['mix_W'], p['mix_b'], h, bm=2000)


def _bn(h, g, b, relu):
    mu = jnp.mean(h, axis=0, keepdims=True)
    var = jnp.var(h, axis=0, keepdims=True)
    return _bn_norm(h, g, b, mu, var, relu, bm=2000)


def kernel(x, edge_index, eig, params):
    src, dst = edge_index[0], edge_index[1]
    h = _bn(x @ params['mlp_W1'] + params['mlp_b1'],
            params['mlp_g1'], params['mlp_be1'], relu=True)
    h = _bn(h @ params['mlp_W2'] + params['mlp_b2'],
            params['mlp_g2'], params['mlp_be2'], relu=True)
    h = h @ params['mlp_W3'] + params['mlp_b3']
    for p in params['convs']:
        h = _bn(_conv(h, eig, p, src, dst), p['bn_g'], p['bn_b'], relu=False)
    return jnp.sum(h, axis=0, keepdims=True)


# speedup vs baseline: 2.7190x; 2.3176x over previous
import functools

import jax
import jax.numpy as jnp
from jax import lax
from jax.experimental import pallas as pl
from jax.experimental.pallas import tpu as pltpu
from jax.experimental.pallas import tpu_sc as plsc

DELTA = 2.5
EIG_IDX = 1

N = 10000
E = 320000


def _mm_bias(a, w, b, bm):
    m, k = a.shape
    f = w.shape[1]

    def body(a_ref, w_ref, b_ref, o_ref):
        o_ref[...] = jnp.dot(a_ref[...], w_ref[...],
                             preferred_element_type=jnp.float32) + b_ref[...]

    return pl.pallas_call(
        body,
        grid=(m // bm,),
        in_specs=[pl.BlockSpec((bm, k), lambda i: (i, 0)),
                  pl.BlockSpec((k, f), lambda i: (0, 0)),
                  pl.BlockSpec((f,), lambda i: (0,))],
        out_specs=pl.BlockSpec((bm, f), lambda i: (i, 0)),
        out_shape=jax.ShapeDtypeStruct((m, f), jnp.float32),
    )(a, w, b)


def _mm_bias_leaky_res(a, w, b, h, bm):
    m, k = a.shape
    f = w.shape[1]

    def body(a_ref, w_ref, b_ref, h_ref, o_ref):
        z = jnp.dot(a_ref[...], w_ref[...],
                    preferred_element_type=jnp.float32) + b_ref[...]
        o_ref[...] = jax.nn.leaky_relu(z, negative_slope=0.01) + h_ref[...]

    return pl.pallas_call(
        body,
        grid=(m // bm,),
        in_specs=[pl.BlockSpec((bm, k), lambda i: (i, 0)),
                  pl.BlockSpec((k, f), lambda i: (0, 0)),
                  pl.BlockSpec((f,), lambda i: (0,)),
                  pl.BlockSpec((bm, f), lambda i: (i, 0))],
        out_specs=pl.BlockSpec((bm, f), lambda i: (i, 0)),
        out_shape=jax.ShapeDtypeStruct((m, f), jnp.float32),
    )(a, w, b, h)


def _bn_norm(y, g, b, mu, var, relu, bm):
    m, f = y.shape

    def body(y_ref, g_ref, b_ref, mu_ref, var_ref, o_ref):
        z = (g_ref[...] * (y_ref[...] - mu_ref[...])
             * lax.rsqrt(var_ref[...] + 1e-5) + b_ref[...])
        o_ref[...] = jax.nn.relu(z) if relu else z

    return pl.pallas_call(
        body,
        grid=(m // bm,),
        in_specs=[pl.BlockSpec((bm, f), lambda i: (i, 0)),
                  pl.BlockSpec((f,), lambda i: (0,)),
                  pl.BlockSpec((f,), lambda i: (0,)),
                  pl.BlockSpec((1, f), lambda i: (0, 0)),
                  pl.BlockSpec((1, f), lambda i: (0, 0))],
        out_specs=pl.BlockSpec((bm, f), lambda i: (i, 0)),
        out_shape=jax.ShapeDtypeStruct((m, f), jnp.float32),
    )(y, g, b, mu, var)


def _weighted_msgs(msg, w_av, w_dx, bm):
    m, f = msg.shape

    def body(m_ref, wa_ref, wd_ref, oa_ref, od_ref):
        mv = m_ref[...]
        oa_ref[...] = mv * wa_ref[...]
        od_ref[...] = mv * wd_ref[...]

    return pl.pallas_call(
        body,
        grid=(m // bm,),
        in_specs=[pl.BlockSpec((bm, f), lambda i: (i, 0)),
                  pl.BlockSpec((bm, 1), lambda i: (i, 0)),
                  pl.BlockSpec((bm, 1), lambda i: (i, 0))],
        out_specs=[pl.BlockSpec((bm, f), lambda i: (i, 0)),
                   pl.BlockSpec((bm, f), lambda i: (i, 0))],
        out_shape=[jax.ShapeDtypeStruct((m, f), jnp.float32),
                   jax.ShapeDtypeStruct((m, f), jnp.float32)],
    )(msg, w_av, w_dx)


_SC_INFO = plsc.get_sparse_core_info()
_NC, _NS = _SC_INFO.num_cores, _SC_INFO.num_subcores
_NW = _NC * _NS
_CH = 400


def _edge_features(table, src, dst):
    b_per_w = E // _NW
    mesh = plsc.VectorSubcoreMesh(core_axis_name="c", subcore_axis_name="s")

    @functools.partial(
        pl.kernel, mesh=mesh,
        out_type=jax.ShapeDtypeStruct((E, 256), jnp.float32),
        scratch_types=[
            pltpu.VMEM((_CH,), jnp.int32),
            pltpu.VMEM((_CH,), jnp.int32),
            pltpu.VMEM((_CH, 128), jnp.float32),
            pltpu.VMEM((_CH, 128), jnp.float32),
            pltpu.SemaphoreType.DMA,
            pltpu.SemaphoreType.DMA,
        ],
    )
    def k(table_hbm, src_hbm, dst_hbm, out_hbm, si_v, di_v, rs_v, rd_v, s1, s2):
        wid = lax.axis_index("s") * _NC + lax.axis_index("c")
        base = wid * b_per_w

        def step(j):
            off = base + j * _CH
            pltpu.sync_copy(src_hbm.at[pl.ds(off, _CH)], si_v)
            pltpu.sync_copy(dst_hbm.at[pl.ds(off, _CH)], di_v)
            cp1 = pltpu.async_copy(table_hbm.at[si_v], rs_v, s1)
            cp2 = pltpu.async_copy(table_hbm.at[di_v], rd_v, s2)
            cp1.wait()
            cp2.wait()
            pltpu.sync_copy(rs_v, out_hbm.at[pl.ds(off, _CH), pl.ds(0, 128)])
            pltpu.sync_copy(rd_v, out_hbm.at[pl.ds(off, _CH), pl.ds(128, 128)])

        pl.loop(0, b_per_w // _CH)(step)

    return k(table, src, dst)


def _gather_scalar(table, idx):
    e = idx.shape[0]
    b_per_w = e // _NW
    mesh = plsc.VectorSubcoreMesh(core_axis_name="c", subcore_axis_name="s")

    @functools.partial(
        pl.kernel, mesh=mesh,
        out_type=jax.ShapeDtypeStruct((e,), jnp.float32),
        scratch_types=[
            pltpu.VMEM((_CH,), jnp.int32),
            pltpu.VMEM((_CH,), jnp.float32),
            pltpu.SemaphoreType.DMA,
        ],
    )
    def k(table_hbm, idx_hbm, out_hbm, i_v, v_v, sem):
        wid = lax.axis_index("s") * _NC + lax.axis_index("c")
        base = wid * b_per_w

        def step(j):
            off = base + j * _CH
            pltpu.sync_copy(idx_hbm.at[pl.ds(off, _CH)], i_v)
            pltpu.async_copy(table_hbm.at[i_v], v_v, sem).wait()
            pltpu.sync_copy(v_v, out_hbm.at[pl.ds(off, _CH)])

        pl.loop(0, b_per_w // _CH)(step)

    return k(table, idx)


def _edge_scalars(eig, src, dst):
    eig_col = eig[:, EIG_IDX]
    eig_s = _gather_scalar(eig_col, src)
    eig_d = _gather_scalar(eig_col, dst)
    diff = eig_s - eig_d
    abs_diff = jnp.abs(diff)
    norm = jax.ops.segment_sum(abs_diff, dst, num_segments=N)
    inv = 1.0 / (_gather_scalar(norm, dst) + 1e-30)
    w_av = (abs_diff * inv)[:, None]
    w_dx = (diff * inv)[:, None]
    s_dx = jax.ops.segment_sum(w_dx, dst, num_segments=N)
    deg = jax.ops.segment_sum(jnp.ones_like(abs_diff), dst, num_segments=N)
    amp = (jnp.log(deg + 1.0) / DELTA)[:, None]
    return w_av, w_dx, s_dx, amp


def _conv(h, p, src, dst, w_av, w_dx, s_dx, amp):
    f = _edge_features(h, src, dst)
    msg = _mm_bias(f, p['M_W'], p['M_b'], bm=2000)
    m_av, m_dx = _weighted_msgs(msg, w_av, w_dx, bm=2000)
    h_av = jax.ops.segment_sum(m_av, dst, num_segments=N)
    h_dx = jnp.abs(jax.ops.segment_sum(m_dx, dst, num_segments=N) - s_dx * h)
    h_sum = jax.ops.segment_sum(msg, dst, num_segments=N)
    h_agg = jnp.concatenate([h_av, h_dx, h_sum], axis=-1)
    h_scaled = jnp.concatenate([h_agg, h_agg * amp], axis=-1)
    h_out = jnp.concatenate([h, h_scaled], axis=-1) @ p['U_W'] + p['U_b']
    return _mm_bias_leaky_res(h_out, p['mix_W'], p['mix_b'], h, bm=2000)


def _bn(h, g, b, relu):
    mu = jnp.mean(h, axis=0, keepdims=True)
    var = jnp.var(h, axis=0, keepdims=True)
    return _bn_norm(h, g, b, mu, var, relu, bm=2000)


def kernel(x, edge_index, eig, params):
    src, dst = edge_index[0], edge_index[1]
    h = _bn(x @ params['mlp_W1'] + params['mlp_b1'],
            params['mlp_g1'], params['mlp_be1'], relu=True)
    h = _bn(h @ params['mlp_W2'] + params['mlp_b2'],
            params['mlp_g2'], params['mlp_be2'], relu=True)
    h = h @ params['mlp_W3'] + params['mlp_b3']
    w_av, w_dx, s_dx, amp = _edge_scalars(eig, src, dst)
    for p in params['convs']:
        h = _bn(_conv(h, p, src, dst, w_av, w_dx, s_dx, amp),
                p['bn_g'], p['bn_b'], relu=False)
    return jnp.sum(h, axis=0, keepdims=True)


# speedup vs baseline: 2.7203x; 1.0005x over previous
import functools

import jax
import jax.numpy as jnp
from jax import lax
from jax.experimental import pallas as pl
from jax.experimental.pallas import tpu as pltpu
from jax.experimental.pallas import tpu_sc as plsc

DELTA = 2.5
EIG_IDX = 1

N = 10000
E = 320000


def _mm_bias(a, w, b, bm):
    m, k = a.shape
    f = w.shape[1]

    def body(a_ref, w_ref, b_ref, o_ref):
        o_ref[...] = jnp.dot(a_ref[...], w_ref[...],
                             preferred_element_type=jnp.float32) + b_ref[...]

    return pl.pallas_call(
        body,
        grid=(m // bm,),
        in_specs=[pl.BlockSpec((bm, k), lambda i: (i, 0)),
                  pl.BlockSpec((k, f), lambda i: (0, 0)),
                  pl.BlockSpec((f,), lambda i: (0,))],
        out_specs=pl.BlockSpec((bm, f), lambda i: (i, 0)),
        out_shape=jax.ShapeDtypeStruct((m, f), jnp.float32),
    )(a, w, b)


def _mm_bias_leaky_res(a, w, b, h, bm):
    m, k = a.shape
    f = w.shape[1]

    def body(a_ref, w_ref, b_ref, h_ref, o_ref):
        z = jnp.dot(a_ref[...], w_ref[...],
                    preferred_element_type=jnp.float32) + b_ref[...]
        o_ref[...] = jax.nn.leaky_relu(z, negative_slope=0.01) + h_ref[...]

    return pl.pallas_call(
        body,
        grid=(m // bm,),
        in_specs=[pl.BlockSpec((bm, k), lambda i: (i, 0)),
                  pl.BlockSpec((k, f), lambda i: (0, 0)),
                  pl.BlockSpec((f,), lambda i: (0,)),
                  pl.BlockSpec((bm, f), lambda i: (i, 0))],
        out_specs=pl.BlockSpec((bm, f), lambda i: (i, 0)),
        out_shape=jax.ShapeDtypeStruct((m, f), jnp.float32),
    )(a, w, b, h)


def _bn_norm(y, g, b, mu, var, relu, bm):
    m, f = y.shape

    def body(y_ref, g_ref, b_ref, mu_ref, var_ref, o_ref):
        z = (g_ref[...] * (y_ref[...] - mu_ref[...])
             * lax.rsqrt(var_ref[...] + 1e-5) + b_ref[...])
        o_ref[...] = jax.nn.relu(z) if relu else z

    return pl.pallas_call(
        body,
        grid=(m // bm,),
        in_specs=[pl.BlockSpec((bm, f), lambda i: (i, 0)),
                  pl.BlockSpec((f,), lambda i: (0,)),
                  pl.BlockSpec((f,), lambda i: (0,)),
                  pl.BlockSpec((1, f), lambda i: (0, 0)),
                  pl.BlockSpec((1, f), lambda i: (0, 0))],
        out_specs=pl.BlockSpec((bm, f), lambda i: (i, 0)),
        out_shape=jax.ShapeDtypeStruct((m, f), jnp.float32),
    )(y, g, b, mu, var)


def _weighted_msgs(msg, w_av, w_dx, bm):
    m, f = msg.shape

    def body(m_ref, wa_ref, wd_ref, oa_ref, od_ref):
        mv = m_ref[...]
        oa_ref[...] = mv * wa_ref[...]
        od_ref[...] = mv * wd_ref[...]

    return pl.pallas_call(
        body,
        grid=(m // bm,),
        in_specs=[pl.BlockSpec((bm, f), lambda i: (i, 0)),
                  pl.BlockSpec((bm, 1), lambda i: (i, 0)),
                  pl.BlockSpec((bm, 1), lambda i: (i, 0))],
        out_specs=[pl.BlockSpec((bm, f), lambda i: (i, 0)),
                   pl.BlockSpec((bm, f), lambda i: (i, 0))],
        out_shape=[jax.ShapeDtypeStruct((m, f), jnp.float32),
                   jax.ShapeDtypeStruct((m, f), jnp.float32)],
    )(msg, w_av, w_dx)


_SC_INFO = plsc.get_sparse_core_info()
_NC, _NS = _SC_INFO.num_cores, _SC_INFO.num_subcores
_NW = _NC * _NS
_CH = 400


def _edge_features(table, src, dst):
    b_per_w = E // _NW
    mesh = plsc.VectorSubcoreMesh(core_axis_name="c", subcore_axis_name="s")

    @functools.partial(
        pl.kernel, mesh=mesh,
        out_type=jax.ShapeDtypeStruct((E, 256), jnp.float32),
        scratch_types=[
            pltpu.VMEM((_CH,), jnp.int32),
            pltpu.VMEM((_CH,), jnp.int32),
            pltpu.VMEM((_CH, 128), jnp.float32),
            pltpu.VMEM((_CH, 128), jnp.float32),
            pltpu.SemaphoreType.DMA,
            pltpu.SemaphoreType.DMA,
        ],
    )
    def k(table_hbm, src_hbm, dst_hbm, out_hbm, si_v, di_v, rs_v, rd_v, s1, s2):
        wid = lax.axis_index("s") * _NC + lax.axis_index("c")
        base = wid * b_per_w

        def step(j):
            off = base + j * _CH
            pltpu.sync_copy(src_hbm.at[pl.ds(off, _CH)], si_v)
            pltpu.sync_copy(dst_hbm.at[pl.ds(off, _CH)], di_v)
            cp1 = pltpu.async_copy(table_hbm.at[si_v], rs_v, s1)
            cp2 = pltpu.async_copy(table_hbm.at[di_v], rd_v, s2)
            cp1.wait()
            cp2.wait()
            pltpu.sync_copy(rs_v, out_hbm.at[pl.ds(off, _CH), pl.ds(0, 128)])
            pltpu.sync_copy(rd_v, out_hbm.at[pl.ds(off, _CH), pl.ds(128, 128)])

        pl.loop(0, b_per_w // _CH)(step)

    return k(table, src, dst)


def _gather_scalar(table, idx):
    e = idx.shape[0]
    b_per_w = e // _NW
    mesh = plsc.VectorSubcoreMesh(core_axis_name="c", subcore_axis_name="s")

    @functools.partial(
        pl.kernel, mesh=mesh,
        out_type=jax.ShapeDtypeStruct((e,), jnp.float32),
        scratch_types=[
            pltpu.VMEM((_CH,), jnp.int32),
            pltpu.VMEM((_CH,), jnp.float32),
            pltpu.SemaphoreType.DMA,
        ],
    )
    def k(table_hbm, idx_hbm, out_hbm, i_v, v_v, sem):
        wid = lax.axis_index("s") * _NC + lax.axis_index("c")
        base = wid * b_per_w

        def step(j):
            off = base + j * _CH
            pltpu.sync_copy(idx_hbm.at[pl.ds(off, _CH)], i_v)
            pltpu.async_copy(table_hbm.at[i_v], v_v, sem).wait()
            pltpu.sync_copy(v_v, out_hbm.at[pl.ds(off, _CH)])

        pl.loop(0, b_per_w // _CH)(step)

    return k(table, idx)


def _edge_scalars(eig, src, dst):
    eig_col = eig[:, EIG_IDX]
    eig_s = _gather_scalar(eig_col, src)
    eig_d = _gather_scalar(eig_col, dst)
    diff = eig_s - eig_d
    abs_diff = jnp.abs(diff)
    norm = jax.ops.segment_sum(abs_diff, dst, num_segments=N)
    inv = 1.0 / (_gather_scalar(norm, dst) + 1e-30)
    w_av = (abs_diff * inv)[:, None]
    w_dx = (diff * inv)[:, None]
    s_dx = jax.ops.segment_sum(w_dx, dst, num_segments=N)
    deg = jax.ops.segment_sum(jnp.ones_like(abs_diff), dst, num_segments=N)
    amp = (jnp.log(deg + 1.0) / DELTA)[:, None]
    return w_av, w_dx, s_dx, amp


def _conv(h, p, src, dst, w_av, w_dx, s_dx, amp):
    f = _edge_features(h, src, dst)
    msg = _mm_bias(f, p['M_W'], p['M_b'], bm=2000)
    h_sum = jax.ops.segment_sum(msg, dst, num_segments=N)
    m_av, m_dx = _weighted_msgs(msg, w_av, w_dx, bm=2000)
    h_av = jax.ops.segment_sum(m_av, dst, num_segments=N)
    h_dx = jnp.abs(jax.ops.segment_sum(m_dx, dst, num_segments=N) - s_dx * h)
    h_agg = jnp.concatenate([h_av, h_dx, h_sum], axis=-1)
    h_scaled = jnp.concatenate([h_agg, h_agg * amp], axis=-1)
    h_out = jnp.concatenate([h, h_scaled], axis=-1) @ p['U_W'] + p['U_b']
    return _mm_bias_leaky_res(h_out, p['mix_W'], p['mix_b'], h, bm=2000)


def _bn(h, g, b, relu):
    mu = jnp.mean(h, axis=0, keepdims=True)
    var = jnp.var(h, axis=0, keepdims=True)
    return _bn_norm(h, g, b, mu, var, relu, bm=2000)


def kernel(x, edge_index, eig, params):
    src, dst = edge_index[0], edge_index[1]
    h = _bn(x @ params['mlp_W1'] + params['mlp_b1'],
            params['mlp_g1'], params['mlp_be1'], relu=True)
    h = _bn(h @ params['mlp_W2'] + params['mlp_b2'],
            params['mlp_g2'], params['mlp_be2'], relu=True)
    h = h @ params['mlp_W3'] + params['mlp_b3']
    w_av, w_dx, s_dx, amp = _edge_scalars(eig, src, dst)
    for p in params['convs']:
        h = _bn(_conv(h, p, src, dst, w_av, w_dx, s_dx, amp),
                p['bn_g'], p['bn_b'], relu=False)
    return jnp.sum(h, axis=0, keepdims=True)
